# baseline (speedup 1.0000x reference)
import functools

import jax
import jax.numpy as jnp
from jax import lax
from jax.experimental import pallas as pl
from jax.experimental.pallas import tpu as pltpu
from jax.experimental.pallas import tpu_sc as plsc

N = 10000
E = 320000
HEADS = 8
DH = 32
HD = HEADS * DH
HALF = HD // 2
NP = 10240
BM = 1024
NSC = 2
NTILE = 16
NW = NSC * NTILE
EPW = E // NW
BB = 40
NB = EPW // BB
RPT = NP // NTILE
ZR = 160


def _tc1_body(x_ref, w_ref, alm_ref, arm_ref,
              ha_ref, hb_ref, el_ref, er_ref, mx_ref):
    i = pl.program_id(0)
    h = jnp.dot(x_ref[...], w_ref[...], preferred_element_type=jnp.float32)
    ha_ref[...] = h[:, :HALF]
    hb_ref[...] = h[:, HALF:]
    el = jnp.dot(h, alm_ref[...], preferred_element_type=jnp.float32)
    er = jnp.dot(h, arm_ref[...], preferred_element_type=jnp.float32)
    el_ref[...] = el
    er_ref[...] = er
    blk = jnp.concatenate(
        [jnp.max(el, 0, keepdims=True), jnp.max(er, 0, keepdims=True),
         jnp.full((6, 16), -jnp.inf, jnp.float32)], 0)

    @pl.when(i == 0)
    def _():
        mx_ref[...] = blk

    @pl.when(i > 0)
    def _():
        mx_ref[...] = jnp.maximum(mx_ref[...], blk)


def _prologue(na_ref, nb_ref, dn_ref, rep_ref, b_ref):
    num = jnp.concatenate([na_ref[0] + na_ref[1], nb_ref[0] + nb_ref[1]], 1)
    den = dn_ref[0] + dn_ref[1]
    denb = jnp.dot(den, rep_ref[...], preferred_element_type=jnp.float32)
    return num / (denb + 1e-9) + b_ref[...]


def _tc2_body(na_ref, nb_ref, dn_ref, rep_ref, b_ref, w_ref, alm_ref, arm_ref,
              ha_ref, hb_ref, el_ref, er_ref, mx_ref):
    i = pl.program_id(0)
    a = jnp.maximum(_prologue(na_ref, nb_ref, dn_ref, rep_ref, b_ref), 0.0)
    h = jnp.dot(a, w_ref[...], preferred_element_type=jnp.float32)
    ha_ref[...] = h[:, :HALF]
    hb_ref[...] = h[:, HALF:]
    el = jnp.dot(h, alm_ref[...], preferred_element_type=jnp.float32)
    er = jnp.dot(h, arm_ref[...], preferred_element_type=jnp.float32)
    el_ref[...] = el
    er_ref[...] = er
    blk = jnp.concatenate(
        [jnp.max(el, 0, keepdims=True), jnp.max(er, 0, keepdims=True),
         jnp.full((6, 16), -jnp.inf, jnp.float32)], 0)

    @pl.when(i == 0)
    def _():
        mx_ref[...] = blk

    @pl.when(i > 0)
    def _():
        mx_ref[...] = jnp.maximum(mx_ref[...], blk)


def _ep_body(na_ref, nb_ref, dn_ref, rep_ref, b_ref, o_ref):
    o_ref[...] = _prologue(na_ref, nb_ref, dn_ref, rep_ref, b_ref)


_HSPECS = [pl.BlockSpec((BM, HALF), lambda i: (i, 0)),
           pl.BlockSpec((BM, HALF), lambda i: (i, 0)),
           pl.BlockSpec((BM, 16), lambda i: (i, 0)),
           pl.BlockSpec((BM, 16), lambda i: (i, 0)),
           pl.BlockSpec((8, 16), lambda i: (0, 0))]
_HSHAPES = (jax.ShapeDtypeStruct((NP, HALF), jnp.float32),
            jax.ShapeDtypeStruct((NP, HALF), jnp.float32),
            jax.ShapeDtypeStruct((NP, 16), jnp.float32),
            jax.ShapeDtypeStruct((NP, 16), jnp.float32),
            jax.ShapeDtypeStruct((8, 16), jnp.float32))
_NSPECS = [pl.BlockSpec((NSC, BM, HALF), lambda i: (0, i, 0)),
           pl.BlockSpec((NSC, BM, HALF), lambda i: (0, i, 0)),
           pl.BlockSpec((NSC, BM, 16), lambda i: (0, i, 0)),
           pl.BlockSpec((16, HD), lambda i: (0, 0)),
           pl.BlockSpec((1, HD), lambda i: (0, 0))]


def _tc1(xp, w, alm, arm):
    return pl.pallas_call(
        _tc1_body,
        grid=(NP // BM,),
        in_specs=[pl.BlockSpec((BM, xp.shape[1]), lambda i: (i, 0)),
                  pl.BlockSpec(w.shape, lambda i: (0, 0)),
                  pl.BlockSpec((HD, 16), lambda i: (0, 0)),
                  pl.BlockSpec((HD, 16), lambda i: (0, 0))],
        out_specs=_HSPECS,
        out_shape=_HSHAPES,
    )(xp, w, alm, arm)


def _tc2(na, nb, dn, rep, b2d, w, alm, arm):
    return pl.pallas_call(
        _tc2_body,
        grid=(NP // BM,),
        in_specs=_NSPECS + [pl.BlockSpec((HD, HD), lambda i: (0, 0)),
                            pl.BlockSpec((HD, 16), lambda i: (0, 0)),
                            pl.BlockSpec((HD, 16), lambda i: (0, 0))],
        out_specs=_HSPECS,
        out_shape=_HSHAPES,
    )(na, nb, dn, rep, b2d, w, alm, arm)


def _epilogue(na, nb, dn, rep, b2d):
    return pl.pallas_call(
        _ep_body,
        grid=(NP // BM,),
        in_specs=_NSPECS,
        out_specs=pl.BlockSpec((BM, HD), lambda i: (i, 0)),
        out_shape=jax.ShapeDtypeStruct((NP, HD), jnp.float32),
    )(na, nb, dn, rep, b2d)


def _sweep_body(sweep, do_den,
                h_hbm, er_hbm, src_hbm, dst_hbm, m_hbm, z_hbm, z2_hbm,
                num_hbm, den_hbm,
                num_sh, den_sh, sidx, didx, dscat, err, hrows, exb, msgb,
                mbuf, isem0, isem1, gsem0, gsem1, ssem0, ssem1):
    c = lax.axis_index("c")
    s = lax.axis_index("s")
    wid = c * NTILE + s
    pltpu.sync_copy(z_hbm, num_sh.at[pl.ds(s * RPT, RPT)])
    if do_den:
        pltpu.sync_copy(z2_hbm, den_sh.at[pl.ds(s * RPT, RPT)])
    pltpu.sync_copy(m_hbm, mbuf)
    plsc.subcore_barrier()
    mvec = mbuf[...]
    base0 = wid * EPW
    isem = (isem0, isem1)
    gsem = (gsem0, gsem1)
    ssem = (ssem0, ssem1)

    def idx_copies(i, si):
        base = base0 + i * BB
        return ((src_hbm.at[pl.ds(base, BB)], sidx.at[si], isem[si]),
                (dst_hbm.at[pl.ds(base, BB)], didx.at[si], isem[si]))

    def gather_copies(si):
        return ((er_hbm.at[didx.at[si]], err.at[si], gsem[si]),
                (h_hbm.at[sidx.at[si]], hrows.at[si], gsem[si]))

    def scatter_copies(si):
        cps = [(msgb.at[si], num_sh.at[dscat.at[si]], ssem[si])]
        if do_den:
            cps.append((exb.at[si], den_sh.at[dscat.at[si]], ssem[si]))
        return cps

    def issue(cps, add=False):
        for src, dst, sem in cps:
            pltpu.async_copy(src, dst, sem, add=add)

    def drain(cps):
        for src, dst, sem in cps:
            pltpu.make_async_copy(src, dst, sem).wait()

    def compute(i, si):
        for off in (0, 16, 24):
            dscat[si, pl.ds(off, 16)] = didx[si, pl.ds(off, 16)]

        @plsc.parallel_loop(0, BB, unroll=4)
        def _edge(e):
            ehi, elo = plsc.unpack(hrows[si, e, pl.ds(HALF, 32)],
                                   format=plsc.PackFormat.INTERLEAVED)
            t = (ehi + elo) + err[si, e, :]
            t = jnp.maximum(t, 0.2 * t)
            ex = jnp.exp(t - mvec)
            exb[si, e, :] = ex
            for hh in range(4):
                sc_ = ex[4 * sweep + hh]
                lo, hi = plsc.unpack(hrows[si, e, pl.ds(hh * DH, 32)],
                                     format=plsc.PackFormat.INTERLEAVED)
                msgb[si, e, pl.ds(hh * DH, 16)] = sc_ * lo
                msgb[si, e, pl.ds(hh * DH + 16, 16)] = sc_ * hi

    def step(i, si, do_a, do_c, do_f):
        ni = 1 - si
        if do_a:
            drain(idx_copies(i + 1, ni))
            issue(gather_copies(ni))
        drain(gather_copies(si))
        if do_c:
            drain(scatter_copies(si))
        compute(i, si)
        issue(scatter_copies(si), add=True)
        if do_f:
            issue(idx_copies(i + 2, si))

    issue(idx_copies(0, 0))
    issue(idx_copies(1, 1))
    drain(idx_copies(0, 0))
    issue(gather_copies(0))
    step(0, 0, True, False, True)
    step(1, 1, True, False, True)

    def pair(k, cc):
        i0 = 2 * k
        step(i0, 0, True, True, True)
        step(i0 + 1, 1, True, True, True)
        return cc

    lax.fori_loop(1, NB // 2 - 1, pair, 0)
    step(NB - 2, 0, True, True, False)
    step(NB - 1, 1, False, True, False)
    drain(scatter_copies(0))
    drain(scatter_copies(1))
    plsc.subcore_barrier()
    r = s * RPT
    pltpu.sync_copy(num_sh.at[pl.ds(r, RPT)], num_hbm.at[c, pl.ds(r, RPT)])
    if do_den:
        pltpu.sync_copy(den_sh.at[pl.ds(r, RPT)], den_hbm.at[c, pl.ds(r, RPT)])


def _make_sweep(sweep, do_den):
    return pl.kernel(
        functools.partial(_sweep_body, sweep, do_den),
        out_type=(jax.ShapeDtypeStruct((NSC, NP, HALF), jnp.float32),
                  jax.ShapeDtypeStruct((NSC, NP, 16), jnp.float32)),
        mesh=plsc.VectorSubcoreMesh(core_axis_name="c", subcore_axis_name="s"),
        compiler_params=pltpu.CompilerParams(use_tc_tiling_on_sc=False,
                                             needs_layout_passes=False),
        scratch_types=[
            pltpu.VMEM_SHARED((NP, HALF), jnp.float32),
            pltpu.VMEM_SHARED((NP, 16), jnp.float32),
            pltpu.VMEM((2, BB), jnp.int32),
            pltpu.VMEM((2, BB), jnp.int32),
            pltpu.VMEM((2, BB), jnp.int32),
            pltpu.VMEM((2, BB, 16), jnp.float32),
            pltpu.VMEM((2, BB, 160), jnp.bfloat16),
            pltpu.VMEM((2, BB, 16), jnp.float32),
            pltpu.VMEM((2, BB, HALF), jnp.float32),
            pltpu.VMEM((16,), jnp.float32),
            pltpu.SemaphoreType.DMA,
            pltpu.SemaphoreType.DMA,
            pltpu.SemaphoreType.DMA,
            pltpu.SemaphoreType.DMA,
            pltpu.SemaphoreType.DMA,
            pltpu.SemaphoreType.DMA,
        ],
    )


_sweepA = _make_sweep(0, True)
_sweepB = _make_sweep(1, False)


def _pack_h(h_half, elpk):
    hb = h_half.astype(jnp.bfloat16).reshape(NP, 4, 2, 16)
    pairs = jnp.stack([hb[:, :, 0, :], hb[:, :, 1, :]], -1)
    hil = pairs.reshape(NP, HALF)
    return jnp.concatenate([hil, elpk], axis=1)


def _pack_el(el):
    hi = el.astype(jnp.bfloat16)
    lo = (el - hi.astype(jnp.float32)).astype(jnp.bfloat16)
    return jnp.stack([hi, lo], -1).reshape(NP, 32)


def _attn_mat(a):
    idx = jnp.arange(HD, dtype=jnp.int32)
    return jnp.zeros((HD, 16), jnp.float32).at[idx, idx // DH].set(a.reshape(-1))


def _mvec(mx):
    return jnp.concatenate([mx[0, :8] + mx[1, :8],
                            jnp.full((8,), 1e30, jnp.float32)])


def kernel(x, edge_index, W1, al1, ar1, b1, W2, al2, ar2, b2):
    src = edge_index[0]
    dst = edge_index[1]
    xp = jnp.pad(x, ((0, NP - N), (0, 0)))
    rep = (jnp.arange(HD)[None, :] // DH ==
           jnp.arange(16)[:, None]).astype(jnp.float32)
    z = jnp.zeros((RPT, HALF), jnp.float32)
    z2 = jnp.zeros((RPT, 16), jnp.float32)

    ha1, hb1, el1, er1, mx1 = _tc1(xp, W1, _attn_mat(al1), _attn_mat(ar1))
    m1 = _mvec(mx1)
    elpk1 = _pack_el(el1)
    na1, dn1 = _sweepA(_pack_h(ha1, elpk1), er1, src, dst, m1, z, z2)
    nb1, _ = _sweepB(_pack_h(hb1, elpk1), er1, src, dst, m1, z, z2)

    ha2, hb2, el2, er2, mx2 = _tc2(na1, nb1, dn1, rep, b1[None, :], W2,
                                   _attn_mat(al2), _attn_mat(ar2))
    m2 = _mvec(mx2)
    elpk2 = _pack_el(el2)
    na2, dn2 = _sweepA(_pack_h(ha2, elpk2), er2, src, dst, m2, z, z2)
    nb2, _ = _sweepB(_pack_h(hb2, elpk2), er2, src, dst, m2, z, z2)

    out = _epilogue(na2, nb2, dn2, rep, b2[None, :])
    return out[:N]


# speedup vs baseline: 1.0894x; 1.0894x over previous
import functools

import jax
import jax.numpy as jnp
from jax import lax
from jax.experimental import pallas as pl
from jax.experimental.pallas import tpu as pltpu
from jax.experimental.pallas import tpu_sc as plsc

N = 10000
E = 320000
HEADS = 8
DH = 32
HD = HEADS * DH
HALF = HD // 2
NP = 10240
BM = 1024
NSC = 2
NTILE = 16
NW = NSC * NTILE
EPW = E // NW
BB = 40
NB = EPW // BB
RPT = NP // NTILE
ZR = 160


def _tc_tail(i, h, alm_ref, arm_ref, em_ref, om_ref,
             ha_ref, hb_ref, er_ref, mx_ref):
    el = jnp.dot(h, alm_ref[...], preferred_element_type=jnp.float32)
    er = jnp.dot(h, arm_ref[...], preferred_element_type=jnp.float32)
    er_ref[...] = er
    hi = el.astype(jnp.bfloat16).astype(jnp.float32)
    lo = el - hi
    el_il = (jnp.dot(hi, em_ref[...], preferred_element_type=jnp.float32) +
             jnp.dot(lo, om_ref[...], preferred_element_type=jnp.float32))
    elb = el_il.astype(jnp.bfloat16)
    ha_ref[...] = jnp.concatenate([h[:, :HALF].astype(jnp.bfloat16), elb], 1)
    hb_ref[...] = jnp.concatenate([h[:, HALF:].astype(jnp.bfloat16), elb], 1)
    blk = jnp.concatenate(
        [jnp.max(el, 0, keepdims=True), jnp.max(er, 0, keepdims=True),
         jnp.full((6, 16), -jnp.inf, jnp.float32)], 0)

    @pl.when(i == 0)
    def _():
        mx_ref[...] = blk

    @pl.when(i > 0)
    def _():
        mx_ref[...] = jnp.maximum(mx_ref[...], blk)


def _tc1_body(x_ref, w_ref, alm_ref, arm_ref, em_ref, om_ref,
              ha_ref, hb_ref, er_ref, mx_ref):
    i = pl.program_id(0)
    h = jnp.dot(x_ref[...], w_ref[...], preferred_element_type=jnp.float32)
    _tc_tail(i, h, alm_ref, arm_ref, em_ref, om_ref,
             ha_ref, hb_ref, er_ref, mx_ref)


def _prologue(na_ref, nb_ref, dn_ref, rep_ref, b_ref):
    num = jnp.concatenate([na_ref[0] + na_ref[1], nb_ref[0] + nb_ref[1]], 1)
    den = dn_ref[0] + dn_ref[1]
    denb = jnp.dot(den, rep_ref[...], preferred_element_type=jnp.float32)
    return num / (denb + 1e-9) + b_ref[...]


def _tc2_body(na_ref, nb_ref, dn_ref, rep_ref, b_ref, w_ref, alm_ref, arm_ref,
              em_ref, om_ref, ha_ref, hb_ref, er_ref, mx_ref):
    i = pl.program_id(0)
    a = jnp.maximum(_prologue(na_ref, nb_ref, dn_ref, rep_ref, b_ref), 0.0)
    h = jnp.dot(a, w_ref[...], preferred_element_type=jnp.float32)
    _tc_tail(i, h, alm_ref, arm_ref, em_ref, om_ref,
             ha_ref, hb_ref, er_ref, mx_ref)


def _ep_body(na_ref, nb_ref, dn_ref, rep_ref, b_ref, o_ref):
    o_ref[...] = _prologue(na_ref, nb_ref, dn_ref, rep_ref, b_ref)


_HSPECS = [pl.BlockSpec((BM, 160), lambda i: (i, 0)),
           pl.BlockSpec((BM, 160), lambda i: (i, 0)),
           pl.BlockSpec((BM, 16), lambda i: (i, 0)),
           pl.BlockSpec((8, 16), lambda i: (0, 0))]
_HSHAPES = (jax.ShapeDtypeStruct((NP, 160), jnp.bfloat16),
            jax.ShapeDtypeStruct((NP, 160), jnp.bfloat16),
            jax.ShapeDtypeStruct((NP, 16), jnp.float32),
            jax.ShapeDtypeStruct((8, 16), jnp.float32))
_NSPECS = [pl.BlockSpec((NSC, BM, HALF), lambda i: (0, i, 0)),
           pl.BlockSpec((NSC, BM, HALF), lambda i: (0, i, 0)),
           pl.BlockSpec((NSC, BM, 16), lambda i: (0, i, 0)),
           pl.BlockSpec((16, HD), lambda i: (0, 0)),
           pl.BlockSpec((1, HD), lambda i: (0, 0))]


_EOSPECS = [pl.BlockSpec((16, 32), lambda i: (0, 0)),
            pl.BlockSpec((16, 32), lambda i: (0, 0))]


def _tc1(xp, w, alm, arm, em, om):
    return pl.pallas_call(
        _tc1_body,
        grid=(NP // BM,),
        in_specs=[pl.BlockSpec((BM, xp.shape[1]), lambda i: (i, 0)),
                  pl.BlockSpec(w.shape, lambda i: (0, 0)),
                  pl.BlockSpec((HD, 16), lambda i: (0, 0)),
                  pl.BlockSpec((HD, 16), lambda i: (0, 0))] + _EOSPECS,
        out_specs=_HSPECS,
        out_shape=_HSHAPES,
    )(xp, w, alm, arm, em, om)


def _tc2(na, nb, dn, rep, b2d, w, alm, arm, em, om):
    return pl.pallas_call(
        _tc2_body,
        grid=(NP // BM,),
        in_specs=_NSPECS + [pl.BlockSpec((HD, HD), lambda i: (0, 0)),
                            pl.BlockSpec((HD, 16), lambda i: (0, 0)),
                            pl.BlockSpec((HD, 16), lambda i: (0, 0))] + _EOSPECS,
        out_specs=_HSPECS,
        out_shape=_HSHAPES,
    )(na, nb, dn, rep, b2d, w, alm, arm, em, om)


def _epilogue(na, nb, dn, rep, b2d):
    return pl.pallas_call(
        _ep_body,
        grid=(NP // BM,),
        in_specs=_NSPECS,
        out_specs=pl.BlockSpec((BM, HD), lambda i: (i, 0)),
        out_shape=jax.ShapeDtypeStruct((NP, HD), jnp.float32),
    )(na, nb, dn, rep, b2d)


def _sweep_body(sweep, do_den,
                h_hbm, er_hbm, src_hbm, dst_hbm, m_hbm, z_hbm, z2_hbm,
                num_hbm, den_hbm,
                num_sh, den_sh, sidx, didx, dscat, err, hrows, exb, msgb,
                mbuf, isem0, isem1, gsem0, gsem1, ssem0, ssem1):
    c = lax.axis_index("c")
    s = lax.axis_index("s")
    wid = c * NTILE + s
    pltpu.sync_copy(z_hbm, num_sh.at[pl.ds(s * RPT, RPT)])
    if do_den:
        pltpu.sync_copy(z2_hbm, den_sh.at[pl.ds(s * RPT, RPT)])
    pltpu.sync_copy(m_hbm, mbuf)
    plsc.subcore_barrier()
    mvec = mbuf[...]
    base0 = wid * EPW
    isem = (isem0, isem1)
    gsem = (gsem0, gsem1)
    ssem = (ssem0, ssem1)

    def idx_copies(i, si):
        base = base0 + i * BB
        return ((src_hbm.at[pl.ds(base, BB)], sidx.at[si], isem[si]),
                (dst_hbm.at[pl.ds(base, BB)], didx.at[si], isem[si]))

    def gather_copies(si):
        return ((er_hbm.at[didx.at[si]], err.at[si], gsem[si]),
                (h_hbm.at[sidx.at[si]], hrows.at[si], gsem[si]))

    def scatter_copies(si):
        cps = [(msgb.at[si], num_sh.at[dscat.at[si]], ssem[si])]
        if do_den:
            cps.append((exb.at[si], den_sh.at[dscat.at[si]], ssem[si]))
        return cps

    def issue(cps, add=False):
        for src, dst, sem in cps:
            pltpu.async_copy(src, dst, sem, add=add)

    def drain(cps):
        for src, dst, sem in cps:
            pltpu.make_async_copy(src, dst, sem).wait()

    def compute(i, si):
        for off in (0, 16, 24):
            dscat[si, pl.ds(off, 16)] = didx[si, pl.ds(off, 16)]

        @plsc.parallel_loop(0, BB, unroll=4)
        def _edge(e):
            ehi, elo = plsc.unpack(hrows[si, e, pl.ds(HALF, 32)],
                                   format=plsc.PackFormat.INTERLEAVED)
            t = (ehi + elo) + err[si, e, :]
            t = jnp.maximum(t, 0.2 * t)
            ex = jnp.exp(t - mvec)
            exb[si, e, :] = ex
            for hh in range(4):
                sc_ = ex[4 * sweep + hh]
                lo, hi = plsc.unpack(hrows[si, e, pl.ds(hh * DH, 32)],
                                     format=plsc.PackFormat.INTERLEAVED)
                msgb[si, e, pl.ds(hh * DH, 16)] = sc_ * lo
                msgb[si, e, pl.ds(hh * DH + 16, 16)] = sc_ * hi

    def step(i, si, do_a, do_c, do_f):
        ni = 1 - si
        if do_a:
            drain(idx_copies(i + 1, ni))
            issue(gather_copies(ni))
        drain(gather_copies(si))
        if do_c:
            drain(scatter_copies(si))
        compute(i, si)
        issue(scatter_copies(si), add=True)
        if do_f:
            issue(idx_copies(i + 2, si))

    issue(idx_copies(0, 0))
    issue(idx_copies(1, 1))
    drain(idx_copies(0, 0))
    issue(gather_copies(0))
    step(0, 0, True, False, True)
    step(1, 1, True, False, True)

    def pair(k, cc):
        i0 = 2 * k
        step(i0, 0, True, True, True)
        step(i0 + 1, 1, True, True, True)
        return cc

    lax.fori_loop(1, NB // 2 - 1, pair, 0)
    step(NB - 2, 0, True, True, False)
    step(NB - 1, 1, False, True, False)
    drain(scatter_copies(0))
    drain(scatter_copies(1))
    plsc.subcore_barrier()
    r = s * RPT
    pltpu.sync_copy(num_sh.at[pl.ds(r, RPT)], num_hbm.at[c, pl.ds(r, RPT)])
    if do_den:
        pltpu.sync_copy(den_sh.at[pl.ds(r, RPT)], den_hbm.at[c, pl.ds(r, RPT)])


def _make_sweep(sweep, do_den):
    return pl.kernel(
        functools.partial(_sweep_body, sweep, do_den),
        out_type=(jax.ShapeDtypeStruct((NSC, NP, HALF), jnp.float32),
                  jax.ShapeDtypeStruct((NSC, NP, 16), jnp.float32)),
        mesh=plsc.VectorSubcoreMesh(core_axis_name="c", subcore_axis_name="s"),
        compiler_params=pltpu.CompilerParams(use_tc_tiling_on_sc=False,
                                             needs_layout_passes=False),
        scratch_types=[
            pltpu.VMEM_SHARED((NP, HALF), jnp.float32),
            pltpu.VMEM_SHARED((NP, 16), jnp.float32),
            pltpu.VMEM((2, BB), jnp.int32),
            pltpu.VMEM((2, BB), jnp.int32),
            pltpu.VMEM((2, BB), jnp.int32),
            pltpu.VMEM((2, BB, 16), jnp.float32),
            pltpu.VMEM((2, BB, 160), jnp.bfloat16),
            pltpu.VMEM((2, BB, 16), jnp.float32),
            pltpu.VMEM((2, BB, HALF), jnp.float32),
            pltpu.VMEM((16,), jnp.float32),
            pltpu.SemaphoreType.DMA,
            pltpu.SemaphoreType.DMA,
            pltpu.SemaphoreType.DMA,
            pltpu.SemaphoreType.DMA,
            pltpu.SemaphoreType.DMA,
            pltpu.SemaphoreType.DMA,
        ],
    )


_sweepA = _make_sweep(0, True)
_sweepB = _make_sweep(1, False)


def _attn_mat(a):
    idx = jnp.arange(HD, dtype=jnp.int32)
    return jnp.zeros((HD, 16), jnp.float32).at[idx, idx // DH].set(a.reshape(-1))


def _mvec(mx):
    return jnp.concatenate([mx[0, :8] + mx[1, :8],
                            jnp.full((8,), 1e30, jnp.float32)])


def kernel(x, edge_index, W1, al1, ar1, b1, W2, al2, ar2, b2):
    src = edge_index[0]
    dst = edge_index[1]
    xp = jnp.pad(x, ((0, NP - N), (0, 0)))
    rep = (jnp.arange(HD)[None, :] // DH ==
           jnp.arange(16)[:, None]).astype(jnp.float32)
    z = jnp.zeros((RPT, HALF), jnp.float32)
    z2 = jnp.zeros((RPT, 16), jnp.float32)
    cc = jnp.arange(HD)
    perm = (cc // DH) * DH + (cc % 2) * 16 + (cc % DH) // 2
    k16 = jnp.arange(16)
    em = (jnp.arange(32)[None, :] == 2 * k16[:, None]).astype(jnp.float32)
    om = (jnp.arange(32)[None, :] == 2 * k16[:, None] + 1).astype(jnp.float32)

    ha1, hb1, er1, mx1 = _tc1(xp, W1[:, perm], _attn_mat(al1)[perm],
                              _attn_mat(ar1)[perm], em, om)
    m1 = _mvec(mx1)
    na1, dn1 = _sweepA(ha1, er1, src, dst, m1, z, z2)
    nb1, _ = _sweepB(hb1, er1, src, dst, m1, z, z2)

    ha2, hb2, er2, mx2 = _tc2(na1, nb1, dn1, rep, b1[None, :], W2[:, perm],
                              _attn_mat(al2)[perm], _attn_mat(ar2)[perm],
                              em, om)
    m2 = _mvec(mx2)
    na2, dn2 = _sweepA(ha2, er2, src, dst, m2, z, z2)
    nb2, _ = _sweepB(hb2, er2, src, dst, m2, z, z2)

    out = _epilogue(na2, nb2, dn2, rep, b2[None, :])
    return out[:N]


# speedup vs baseline: 1.0896x; 1.0002x over previous
import functools

import jax
import jax.numpy as jnp
from jax import lax
from jax.experimental import pallas as pl
from jax.experimental.pallas import tpu as pltpu
from jax.experimental.pallas import tpu_sc as plsc

N = 10000
E = 320000
HEADS = 8
DH = 32
HD = HEADS * DH
HALF = HD // 2
NP = 10240
BM = 1024
NSC = 2
NTILE = 16
NW = NSC * NTILE
EPW = E // NW
BB = 40
NB = EPW // BB
RPT = NP // NTILE
ZR = 160


def _tc_tail(i, h, alm_ref, arm_ref, em_ref, om_ref,
             ha_ref, hb_ref, er_ref, mx_ref):
    el = jnp.dot(h, alm_ref[...], preferred_element_type=jnp.float32)
    er = jnp.dot(h, arm_ref[...], preferred_element_type=jnp.float32)
    er_ref[...] = er
    hi = el.astype(jnp.bfloat16).astype(jnp.float32)
    lo = el - hi
    el_il = (jnp.dot(hi, em_ref[...], preferred_element_type=jnp.float32) +
             jnp.dot(lo, om_ref[...], preferred_element_type=jnp.float32))
    elb = el_il.astype(jnp.bfloat16)
    ha_ref[...] = jnp.concatenate([h[:, :HALF].astype(jnp.bfloat16), elb], 1)
    hb_ref[...] = jnp.concatenate([h[:, HALF:].astype(jnp.bfloat16), elb], 1)
    blk = jnp.concatenate(
        [jnp.max(el, 0, keepdims=True), jnp.max(er, 0, keepdims=True),
         jnp.full((6, 16), -jnp.inf, jnp.float32)], 0)

    @pl.when(i == 0)
    def _():
        mx_ref[...] = blk

    @pl.when(i > 0)
    def _():
        mx_ref[...] = jnp.maximum(mx_ref[...], blk)


def _tc1_body(x_ref, w_ref, alm_ref, arm_ref, em_ref, om_ref,
              ha_ref, hb_ref, er_ref, mx_ref):
    i = pl.program_id(0)
    h = jnp.dot(x_ref[...], w_ref[...], preferred_element_type=jnp.float32)
    _tc_tail(i, h, alm_ref, arm_ref, em_ref, om_ref,
             ha_ref, hb_ref, er_ref, mx_ref)


def _prologue(na_ref, nb_ref, rep_ref, b_ref):
    na = na_ref[0] + na_ref[1]
    num = jnp.concatenate([na[:, :HALF], nb_ref[0] + nb_ref[1]], 1)
    den = na[:, HALF:]
    denb = jnp.dot(den, rep_ref[...], preferred_element_type=jnp.float32)
    return num / (denb + 1e-9) + b_ref[...]


def _tc2_body(na_ref, nb_ref, rep_ref, b_ref, w_ref, alm_ref, arm_ref,
              em_ref, om_ref, ha_ref, hb_ref, er_ref, mx_ref):
    i = pl.program_id(0)
    a = jnp.maximum(_prologue(na_ref, nb_ref, rep_ref, b_ref), 0.0)
    h = jnp.dot(a, w_ref[...], preferred_element_type=jnp.float32)
    _tc_tail(i, h, alm_ref, arm_ref, em_ref, om_ref,
             ha_ref, hb_ref, er_ref, mx_ref)


def _ep_body(na_ref, nb_ref, rep_ref, b_ref, o_ref):
    o_ref[...] = _prologue(na_ref, nb_ref, rep_ref, b_ref)


_HSPECS = [pl.BlockSpec((BM, 160), lambda i: (i, 0)),
           pl.BlockSpec((BM, 160), lambda i: (i, 0)),
           pl.BlockSpec((BM, 16), lambda i: (i, 0)),
           pl.BlockSpec((8, 16), lambda i: (0, 0))]
_HSHAPES = (jax.ShapeDtypeStruct((NP, 160), jnp.bfloat16),
            jax.ShapeDtypeStruct((NP, 160), jnp.bfloat16),
            jax.ShapeDtypeStruct((NP, 16), jnp.float32),
            jax.ShapeDtypeStruct((8, 16), jnp.float32))
_NSPECS = [pl.BlockSpec((NSC, BM, 144), lambda i: (0, i, 0)),
           pl.BlockSpec((NSC, BM, HALF), lambda i: (0, i, 0)),
           pl.BlockSpec((16, HD), lambda i: (0, 0)),
           pl.BlockSpec((1, HD), lambda i: (0, 0))]


_EOSPECS = [pl.BlockSpec((16, 32), lambda i: (0, 0)),
            pl.BlockSpec((16, 32), lambda i: (0, 0))]


def _tc1(xp, w, alm, arm, em, om):
    return pl.pallas_call(
        _tc1_body,
        grid=(NP // BM,),
        in_specs=[pl.BlockSpec((BM, xp.shape[1]), lambda i: (i, 0)),
                  pl.BlockSpec(w.shape, lambda i: (0, 0)),
                  pl.BlockSpec((HD, 16), lambda i: (0, 0)),
                  pl.BlockSpec((HD, 16), lambda i: (0, 0))] + _EOSPECS,
        out_specs=_HSPECS,
        out_shape=_HSHAPES,
    )(xp, w, alm, arm, em, om)


def _tc2(na, nb, rep, b2d, w, alm, arm, em, om):
    return pl.pallas_call(
        _tc2_body,
        grid=(NP // BM,),
        in_specs=_NSPECS + [pl.BlockSpec((HD, HD), lambda i: (0, 0)),
                            pl.BlockSpec((HD, 16), lambda i: (0, 0)),
                            pl.BlockSpec((HD, 16), lambda i: (0, 0))] + _EOSPECS,
        out_specs=_HSPECS,
        out_shape=_HSHAPES,
    )(na, nb, rep, b2d, w, alm, arm, em, om)


def _epilogue(na, nb, rep, b2d):
    return pl.pallas_call(
        _ep_body,
        grid=(NP // BM,),
        in_specs=_NSPECS,
        out_specs=pl.BlockSpec((BM, HD), lambda i: (i, 0)),
        out_shape=jax.ShapeDtypeStruct((NP, HD), jnp.float32),
    )(na, nb, rep, b2d)


def _sweep_body(sweep, do_den,
                h_hbm, er_hbm, src_hbm, dst_hbm, m_hbm, z_hbm,
                num_hbm,
                num_sh, sidx, didx, dscat, err, hrows, msgb,
                mbuf, isem0, isem1, gsem0, gsem1, ssem0, ssem1):
    c = lax.axis_index("c")
    s = lax.axis_index("s")
    wid = c * NTILE + s
    pltpu.sync_copy(z_hbm, num_sh.at[pl.ds(s * RPT, RPT)])
    pltpu.sync_copy(m_hbm, mbuf)
    plsc.subcore_barrier()
    mvec = mbuf[...]
    base0 = wid * EPW
    isem = (isem0, isem1)
    gsem = (gsem0, gsem1)
    ssem = (ssem0, ssem1)

    def idx_copies(i, si):
        base = base0 + i * BB
        return ((src_hbm.at[pl.ds(base, BB)], sidx.at[si], isem[si]),
                (dst_hbm.at[pl.ds(base, BB)], didx.at[si], isem[si]))

    def gather_copies(si):
        return ((er_hbm.at[didx.at[si]], err.at[si], gsem[si]),
                (h_hbm.at[sidx.at[si]], hrows.at[si], gsem[si]))

    def scatter_copies(si):
        return ((msgb.at[si], num_sh.at[dscat.at[si]], ssem[si]),)

    def issue(cps, add=False):
        for src, dst, sem in cps:
            pltpu.async_copy(src, dst, sem, add=add)

    def drain(cps):
        for src, dst, sem in cps:
            pltpu.make_async_copy(src, dst, sem).wait()

    def compute(i, si):
        for off in (0, 16, 24):
            dscat[si, pl.ds(off, 16)] = didx[si, pl.ds(off, 16)]

        @plsc.parallel_loop(0, BB, unroll=4)
        def _edge(e):
            ehi, elo = plsc.unpack(hrows[si, e, pl.ds(HALF, 32)],
                                   format=plsc.PackFormat.INTERLEAVED)
            t = (ehi + elo) + err[si, e, :]
            t = jnp.maximum(t, 0.2 * t)
            ex = jnp.exp(t - mvec)
            if do_den:
                msgb[si, e, pl.ds(HALF, 16)] = ex
            for hh in range(4):
                sc_ = ex[4 * sweep + hh]
                lo, hi = plsc.unpack(hrows[si, e, pl.ds(hh * DH, 32)],
                                     format=plsc.PackFormat.INTERLEAVED)
                msgb[si, e, pl.ds(hh * DH, 16)] = sc_ * lo
                msgb[si, e, pl.ds(hh * DH + 16, 16)] = sc_ * hi

    def step(i, si, do_a, do_c, do_f):
        ni = 1 - si
        if do_a:
            drain(idx_copies(i + 1, ni))
            issue(gather_copies(ni))
        drain(gather_copies(si))
        if do_c:
            drain(scatter_copies(si))
        compute(i, si)
        issue(scatter_copies(si), add=True)
        if do_f:
            issue(idx_copies(i + 2, si))

    issue(idx_copies(0, 0))
    issue(idx_copies(1, 1))
    drain(idx_copies(0, 0))
    issue(gather_copies(0))
    step(0, 0, True, False, True)
    step(1, 1, True, False, True)

    def pair(k, cc):
        i0 = 2 * k
        step(i0, 0, True, True, True)
        step(i0 + 1, 1, True, True, True)
        return cc

    lax.fori_loop(1, NB // 2 - 1, pair, 0)
    step(NB - 2, 0, True, True, False)
    step(NB - 1, 1, False, True, False)
    drain(scatter_copies(0))
    drain(scatter_copies(1))
    plsc.subcore_barrier()
    r = s * RPT
    pltpu.sync_copy(num_sh.at[pl.ds(r, RPT)], num_hbm.at[c, pl.ds(r, RPT)])


def _make_sweep(sweep, do_den):
    width = 144 if do_den else HALF
    return pl.kernel(
        functools.partial(_sweep_body, sweep, do_den),
        out_type=jax.ShapeDtypeStruct((NSC, NP, width), jnp.float32),
        mesh=plsc.VectorSubcoreMesh(core_axis_name="c", subcore_axis_name="s"),
        compiler_params=pltpu.CompilerParams(use_tc_tiling_on_sc=False,
                                             needs_layout_passes=False),
        scratch_types=[
            pltpu.VMEM_SHARED((NP, width), jnp.float32),
            pltpu.VMEM((2, BB), jnp.int32),
            pltpu.VMEM((2, BB), jnp.int32),
            pltpu.VMEM((2, BB), jnp.int32),
            pltpu.VMEM((2, BB, 16), jnp.float32),
            pltpu.VMEM((2, BB, 160), jnp.bfloat16),
            pltpu.VMEM((2, BB, width), jnp.float32),
            pltpu.VMEM((16,), jnp.float32),
            pltpu.SemaphoreType.DMA,
            pltpu.SemaphoreType.DMA,
            pltpu.SemaphoreType.DMA,
            pltpu.SemaphoreType.DMA,
            pltpu.SemaphoreType.DMA,
            pltpu.SemaphoreType.DMA,
        ],
    )


_sweepA = _make_sweep(0, True)
_sweepB = _make_sweep(1, False)


def _attn_mat(a):
    idx = jnp.arange(HD, dtype=jnp.int32)
    return jnp.zeros((HD, 16), jnp.float32).at[idx, idx // DH].set(a.reshape(-1))


def _mvec(mx):
    return jnp.concatenate([mx[0, :8] + mx[1, :8],
                            jnp.full((8,), 1e30, jnp.float32)])


def kernel(x, edge_index, W1, al1, ar1, b1, W2, al2, ar2, b2):
    src = edge_index[0]
    dst = edge_index[1]
    xp = jnp.pad(x, ((0, NP - N), (0, 0)))
    rep = (jnp.arange(HD)[None, :] // DH ==
           jnp.arange(16)[:, None]).astype(jnp.float32)
    za = jnp.zeros((RPT, 144), jnp.float32)
    zb = jnp.zeros((RPT, HALF), jnp.float32)
    cc = jnp.arange(HD)
    perm = (cc // DH) * DH + (cc % 2) * 16 + (cc % DH) // 2
    k16 = jnp.arange(16)
    em = (jnp.arange(32)[None, :] == 2 * k16[:, None]).astype(jnp.float32)
    om = (jnp.arange(32)[None, :] == 2 * k16[:, None] + 1).astype(jnp.float32)

    ha1, hb1, er1, mx1 = _tc1(xp, W1[:, perm], _attn_mat(al1)[perm],
                              _attn_mat(ar1)[perm], em, om)
    m1 = _mvec(mx1)
    na1 = _sweepA(ha1, er1, src, dst, m1, za)
    nb1 = _sweepB(hb1, er1, src, dst, m1, zb)

    ha2, hb2, er2, mx2 = _tc2(na1, nb1, rep, b1[None, :], W2[:, perm],
                              _attn_mat(al2)[perm], _attn_mat(ar2)[perm],
                              em, om)
    m2 = _mvec(mx2)
    na2 = _sweepA(ha2, er2, src, dst, m2, za)
    nb2 = _sweepB(hb2, er2, src, dst, m2, zb)

    out = _epilogue(na2, nb2, rep, b2[None, :])
    return out[:N]
